# baseline (speedup 1.0000x reference)
import functools

import jax
import jax.numpy as jnp
from jax import lax
from jax.experimental import pallas as pl
from jax.experimental.pallas import tpu as pltpu
from jax.experimental.pallas import tpu_sc as plsc

N_NODES = 10000
N_EDGES = 320000
NUM_CORES = 2
NUM_SUBCORES = 16
NUM_WORKERS = NUM_CORES * NUM_SUBCORES
EDGES_PER_WORKER = N_EDGES // NUM_WORKERS
CHUNK = 400
NCHUNK = EDGES_PER_WORKER // CHUNK
ROWS_PER_TILE = N_NODES // NUM_SUBCORES
DEG_W = 16

ROW_BLOCK = 1000
N_BLOCKS = N_NODES // ROW_BLOCK

_MESH = plsc.VectorSubcoreMesh(core_axis_name="c", subcore_axis_name="s")
_SC_PARAMS = pltpu.CompilerParams(use_tc_tiling_on_sc=False)


def _sc_degree(edges, ones, zeros):
    @functools.partial(
        pl.kernel,
        out_type=jax.ShapeDtypeStruct((NUM_CORES, N_NODES, DEG_W), jnp.float32),
        mesh=_MESH,
        compiler_params=_SC_PARAMS,
        scratch_types=[
            pltpu.VMEM((EDGES_PER_WORKER,), jnp.int32),
            pltpu.VMEM((CHUNK, DEG_W), jnp.float32),
            pltpu.VMEM_SHARED((N_NODES, DEG_W), jnp.float32),
            pltpu.SemaphoreType.DMA,
        ],
    )
    def k(e_hbm, ones_hbm, zeros_hbm, out_hbm, dstv, onesv, acc, sem):
        c = lax.axis_index("c")
        s = lax.axis_index("s")
        w = c * NUM_SUBCORES + s
        rows = pl.ds(s * ROWS_PER_TILE, ROWS_PER_TILE)
        pltpu.sync_copy(e_hbm.at[1, pl.ds(w * EDGES_PER_WORKER, EDGES_PER_WORKER)], dstv)
        pltpu.sync_copy(ones_hbm, onesv)
        pltpu.sync_copy(zeros_hbm, acc.at[rows])
        plsc.subcore_barrier()

        def idx(i):
            return dstv.at[pl.ds(i * CHUNK, CHUNK)]

        def fire(i, carry):
            pltpu.async_copy(onesv, acc.at[idx(i)], sem, add=True)
            return carry

        lax.fori_loop(0, NCHUNK, fire, 0)

        def drain(i, carry):
            pltpu.make_async_copy(onesv, acc.at[idx(i)], sem).wait()
            return carry

        lax.fori_loop(0, NCHUNK, drain, 0)
        plsc.subcore_barrier()
        pltpu.sync_copy(acc.at[rows], out_hbm.at[c, rows])

    return k(edges, ones, zeros)


def _sc_aggregate(hp, edges, zeros, feat):
    @functools.partial(
        pl.kernel,
        out_type=jax.ShapeDtypeStruct((NUM_CORES, N_NODES, feat), jnp.float32),
        mesh=_MESH,
        compiler_params=_SC_PARAMS,
        scratch_types=[
            pltpu.VMEM((EDGES_PER_WORKER,), jnp.int32),
            pltpu.VMEM((EDGES_PER_WORKER,), jnp.int32),
            [pltpu.VMEM((CHUNK, feat), jnp.float32)] * 2,
            pltpu.VMEM_SHARED((N_NODES, feat), jnp.float32),
            pltpu.VMEM_SHARED((N_NODES, feat), jnp.float32),
            [pltpu.SemaphoreType.DMA] * 2,
            [pltpu.SemaphoreType.DMA] * 2,
        ],
    )
    def k(hp_hbm, e_hbm, zeros_hbm, out_hbm, srcv, dstv, bufs, acc, hps, gsems, ssems):
        c = lax.axis_index("c")
        s = lax.axis_index("s")
        w = c * NUM_SUBCORES + s
        rows = pl.ds(s * ROWS_PER_TILE, ROWS_PER_TILE)
        pltpu.sync_copy(e_hbm.at[0, pl.ds(w * EDGES_PER_WORKER, EDGES_PER_WORKER)], srcv)
        pltpu.sync_copy(e_hbm.at[1, pl.ds(w * EDGES_PER_WORKER, EDGES_PER_WORKER)], dstv)
        pltpu.sync_copy(zeros_hbm, acc.at[rows])
        pltpu.sync_copy(hp_hbm.at[rows], hps.at[rows])
        plsc.subcore_barrier()

        def sidx(i):
            return srcv.at[pl.ds(i * CHUNK, CHUNK)]

        def didx(i):
            return dstv.at[pl.ds(i * CHUNK, CHUNK)]

        def gather(i, b):
            pltpu.async_copy(hps.at[sidx(i)], bufs[b], gsems[b])

        def gwait(i, b):
            pltpu.make_async_copy(hps.at[sidx(i)], bufs[b], gsems[b]).wait()

        def scat(i, b):
            pltpu.async_copy(bufs[b], acc.at[didx(i)], ssems[b], add=True)

        def swait(i, b):
            pltpu.make_async_copy(bufs[b], acc.at[didx(i)], ssems[b]).wait()

        gather(0, 0)
        gwait(0, 0)
        scat(0, 0)
        gather(1, 1)

        def body(kk, carry):
            for j, b in ((1, 1), (2, 0)):
                i = 2 * kk + j
                gwait(i, b)
                scat(i, b)
                swait(i - 1, 1 - b)
                gather(i + 1, 1 - b)
            return carry

        lax.fori_loop(0, (NCHUNK - 3) // 2, body, 0)
        for i in (NCHUNK - 2, NCHUNK - 1):
            b = i % 2
            gwait(i, b)
            scat(i, b)
            swait(i - 1, 1 - b)
            if i + 1 < NCHUNK:
                gather(i + 1, 1 - b)
        swait(NCHUNK - 1, (NCHUNK - 1) % 2)
        plsc.subcore_barrier()
        pltpu.sync_copy(acc.at[rows], out_hbm.at[c, rows])

    return k(hp, edges, zeros)


def _row_spec(d):
    return pl.BlockSpec((ROW_BLOCK, d), lambda i: (i, 0))


def _full_spec(shape):
    return pl.BlockSpec(shape, lambda i: tuple(0 for _ in shape))


def _tc_stage1(x, w, degp):
    def body(x_ref, w_ref, degp_ref, hp_ref, self_ref, dis_ref):
        deg = degp_ref[0, :, 0:1] + degp_ref[1, :, 0:1] + 1.0
        dis = lax.rsqrt(deg)
        h = jnp.dot(x_ref[...], w_ref[...], preferred_element_type=jnp.float32)
        hp_ref[...] = h * dis
        self_ref[...] = h / deg
        dis_ref[...] = dis

    d = w.shape[1]
    return pl.pallas_call(
        body,
        grid=(N_BLOCKS,),
        in_specs=[
            _row_spec(x.shape[1]),
            _full_spec(w.shape),
            pl.BlockSpec((2, ROW_BLOCK, DEG_W), lambda i: (0, i, 0)),
        ],
        out_specs=[
            _row_spec(d),
            _row_spec(d),
            _row_spec(1),
        ],
        out_shape=[
            jax.ShapeDtypeStruct((N_NODES, d), jnp.float32),
            jax.ShapeDtypeStruct((N_NODES, d), jnp.float32),
            jax.ShapeDtypeStruct((N_NODES, 1), jnp.float32),
        ],
    )(x, w, degp)


def _tc_mid(accp, selfp, dis, b, w):
    def body(accp_ref, self_ref, dis_ref, b_ref, w_ref, hp_ref, so_ref):
        dis_ = dis_ref[...]
        z = dis_ * (accp_ref[0] + accp_ref[1]) + self_ref[...] + b_ref[...]
        a = jnp.maximum(z, 0.0)
        h = jnp.dot(a, w_ref[...], preferred_element_type=jnp.float32)
        hp_ref[...] = h * dis_
        so_ref[...] = h * (dis_ * dis_)

    d = selfp.shape[1]
    d2 = w.shape[1]
    return pl.pallas_call(
        body,
        grid=(N_BLOCKS,),
        in_specs=[
            pl.BlockSpec((2, ROW_BLOCK, d), lambda i: (0, i, 0)),
            _row_spec(d),
            _row_spec(1),
            _full_spec(b.shape),
            _full_spec(w.shape),
        ],
        out_specs=[
            _row_spec(d2),
            _row_spec(d2),
        ],
        out_shape=[
            jax.ShapeDtypeStruct((N_NODES, d2), jnp.float32),
            jax.ShapeDtypeStruct((N_NODES, d2), jnp.float32),
        ],
    )(accp, selfp, dis, b, w)


def _tc_final(accp, selfp, dis, b):
    def body(accp_ref, self_ref, dis_ref, b_ref, o_ref):
        z = (dis_ref[...] * (accp_ref[0] + accp_ref[1])
             + self_ref[...] + b_ref[...])
        m = jnp.max(z, axis=1, keepdims=True)
        e = jnp.exp(z - m)
        o_ref[...] = (z - m) - jnp.log(jnp.sum(e, axis=1, keepdims=True))

    d = selfp.shape[1]
    return pl.pallas_call(
        body,
        grid=(N_BLOCKS,),
        in_specs=[
            pl.BlockSpec((2, ROW_BLOCK, d), lambda i: (0, i, 0)),
            _row_spec(d),
            _row_spec(1),
            _full_spec(b.shape),
        ],
        out_specs=_row_spec(d),
        out_shape=jax.ShapeDtypeStruct((N_NODES, d), jnp.float32),
    )(accp, selfp, dis, b)


def kernel(x, edge_index, W1, b1, W2, b2, W3, b3):
    ones = jnp.ones((CHUNK, DEG_W), jnp.float32)

    degp = _sc_degree(edge_index, ones,
                      jnp.zeros((ROWS_PER_TILE, DEG_W), jnp.float32))
    hp1, self1, dis = _tc_stage1(x, W1, degp)

    acc1 = _sc_aggregate(hp1, edge_index,
                         jnp.zeros((ROWS_PER_TILE, 32), jnp.float32), 32)
    hp2, self2 = _tc_mid(acc1, self1, dis, b1.reshape(1, -1), W2)

    acc2 = _sc_aggregate(hp2, edge_index,
                         jnp.zeros((ROWS_PER_TILE, 16), jnp.float32), 16)
    hp3, self3 = _tc_mid(acc2, self2, dis, b2.reshape(1, -1), W3)

    acc3 = _sc_aggregate(hp3, edge_index,
                         jnp.zeros((ROWS_PER_TILE, 40), jnp.float32), 40)
    return _tc_final(acc3, self3, dis, b3.reshape(1, -1))


# speedup vs baseline: 1.0789x; 1.0789x over previous
import functools

import jax
import jax.numpy as jnp
from jax import lax
from jax.experimental import pallas as pl
from jax.experimental.pallas import tpu as pltpu
from jax.experimental.pallas import tpu_sc as plsc

N_NODES = 10000
N_EDGES = 320000
NUM_CORES = 2
NUM_SUBCORES = 16
NUM_WORKERS = NUM_CORES * NUM_SUBCORES
EDGES_PER_WORKER = N_EDGES // NUM_WORKERS
CHUNK = 250
NCHUNK = EDGES_PER_WORKER // CHUNK
ROWS_PER_TILE = N_NODES // NUM_SUBCORES
DEG_W = 16
NBUF = 4

_MESH = plsc.VectorSubcoreMesh(core_axis_name="c", subcore_axis_name="s")
_SC_PARAMS = pltpu.CompilerParams(use_tc_tiling_on_sc=False)


def _sc_degree(e3, ones, zeros):
    @functools.partial(
        pl.kernel,
        out_type=jax.ShapeDtypeStruct((NUM_CORES, N_NODES, DEG_W), jnp.float32),
        mesh=_MESH,
        compiler_params=_SC_PARAMS,
        scratch_types=[
            pltpu.VMEM((NCHUNK, CHUNK), jnp.int32),
            pltpu.VMEM((CHUNK, DEG_W), jnp.float32),
            pltpu.VMEM_SHARED((N_NODES, DEG_W), jnp.float32),
            pltpu.SemaphoreType.DMA,
            [pltpu.SemaphoreType.DMA] * 3,
        ],
    )
    def k(e3_hbm, ones_hbm, zeros_hbm, out_hbm, dstv, onesv, acc, sem, isems):
        c = lax.axis_index("c")
        s = lax.axis_index("s")
        w = c * NUM_SUBCORES + s
        rows = pl.ds(s * ROWS_PER_TILE, ROWS_PER_TILE)
        cp0 = pltpu.async_copy(e3_hbm.at[1, pl.ds(w * NCHUNK, NCHUNK)], dstv,
                               isems[0])
        cp1 = pltpu.async_copy(ones_hbm, onesv, isems[1])
        cp2 = pltpu.async_copy(zeros_hbm, acc.at[rows], isems[2])
        cp0.wait()
        cp1.wait()
        cp2.wait()
        plsc.subcore_barrier()

        def fire(i, carry):
            pltpu.async_copy(onesv, acc.at[dstv.at[i]], sem, add=True)
            return carry

        lax.fori_loop(0, NCHUNK, fire, 0)

        def drain(i, carry):
            pltpu.make_async_copy(onesv, acc.at[dstv.at[i]], sem).wait()
            return carry

        lax.fori_loop(0, NCHUNK, drain, 0)
        plsc.subcore_barrier()
        pltpu.sync_copy(acc.at[rows], out_hbm.at[c, rows])

    return k(e3, ones, zeros)


def _sc_aggregate(hp, e3, zeros, feat):
    @functools.partial(
        pl.kernel,
        out_type=jax.ShapeDtypeStruct((NUM_CORES, N_NODES, feat), jnp.float32),
        mesh=_MESH,
        compiler_params=_SC_PARAMS,
        scratch_types=[
            pltpu.VMEM((NCHUNK, CHUNK), jnp.int32),
            pltpu.VMEM((NCHUNK, CHUNK), jnp.int32),
            [pltpu.VMEM((CHUNK, feat), jnp.float32)] * NBUF,
            pltpu.VMEM_SHARED((N_NODES, feat), jnp.float32),
            pltpu.VMEM_SHARED((N_NODES, feat), jnp.float32),
            [pltpu.SemaphoreType.DMA] * NBUF,
            [pltpu.SemaphoreType.DMA] * NBUF,
        ],
    )
    def k(hp_hbm, e3_hbm, zeros_hbm, out_hbm, srcv, dstv, bufs, acc, hps,
          gsems, ssems):
        c = lax.axis_index("c")
        s = lax.axis_index("s")
        w = c * NUM_SUBCORES + s
        rows = pl.ds(s * ROWS_PER_TILE, ROWS_PER_TILE)
        cp0 = pltpu.async_copy(e3_hbm.at[0, pl.ds(w * NCHUNK, NCHUNK)], srcv,
                               gsems[0])
        cp1 = pltpu.async_copy(e3_hbm.at[1, pl.ds(w * NCHUNK, NCHUNK)], dstv,
                               gsems[1])
        cp2 = pltpu.async_copy(zeros_hbm, acc.at[rows], gsems[2])
        cp3 = pltpu.async_copy(hp_hbm.at[rows], hps.at[rows], gsems[3])
        cp0.wait()
        cp1.wait()
        cp2.wait()
        cp3.wait()
        plsc.subcore_barrier()

        def gather(i, b):
            pltpu.async_copy(hps.at[srcv.at[i]], bufs[b], gsems[b])

        def gwait(i, b):
            pltpu.make_async_copy(hps.at[srcv.at[i]], bufs[b], gsems[b]).wait()

        def scat(i, b):
            pltpu.async_copy(bufs[b], acc.at[dstv.at[i]], ssems[b], add=True)

        def swait(i, b):
            pltpu.make_async_copy(bufs[b], acc.at[dstv.at[i]], ssems[b]).wait()

        gather(0, 0)
        gather(1, 1)
        for i in (0, 1):
            gwait(i, i)
            scat(i, i)
            gather(i + 2, i + 2)

        def body(kk, carry):
            for j in range(NBUF):
                i = NBUF * kk + j + 2
                b = (j + 2) % NBUF
                gwait(i, b)
                scat(i, b)
                swait(i - 2, j)
                gather(i + 2, j)
            return carry

        lax.fori_loop(0, (NCHUNK - 4) // NBUF, body, 0)
        for i in (NCHUNK - 2, NCHUNK - 1):
            b = i % NBUF
            gwait(i, b)
            scat(i, b)
            swait(i - 2, (i + 2) % NBUF)
        swait(NCHUNK - 2, (NCHUNK - 2) % NBUF)
        swait(NCHUNK - 1, (NCHUNK - 1) % NBUF)
        plsc.subcore_barrier()
        pltpu.sync_copy(acc.at[rows], out_hbm.at[c, rows])

    return k(hp, e3, zeros)


def _tc_stage1(x, w, degp):
    def body(x_ref, w_ref, degp_ref, hp_ref, self_ref, dis_ref):
        deg = degp_ref[0, :, 0:1] + degp_ref[1, :, 0:1] + 1.0
        dis = lax.rsqrt(deg)
        h = jnp.dot(x_ref[...], w_ref[...], preferred_element_type=jnp.float32)
        hp_ref[...] = h * dis
        self_ref[...] = h / deg
        dis_ref[...] = dis

    d = w.shape[1]
    return pl.pallas_call(
        body,
        out_shape=[
            jax.ShapeDtypeStruct((N_NODES, d), jnp.float32),
            jax.ShapeDtypeStruct((N_NODES, d), jnp.float32),
            jax.ShapeDtypeStruct((N_NODES, 1), jnp.float32),
        ],
    )(x, w, degp)


def _tc_mid(accp, selfp, dis, b, w):
    def body(accp_ref, self_ref, dis_ref, b_ref, w_ref, hp_ref, so_ref):
        dis_ = dis_ref[...]
        z = dis_ * (accp_ref[0] + accp_ref[1]) + self_ref[...] + b_ref[...]
        a = jnp.maximum(z, 0.0)
        h = jnp.dot(a, w_ref[...], preferred_element_type=jnp.float32)
        hp_ref[...] = h * dis_
        so_ref[...] = h * (dis_ * dis_)

    d2 = w.shape[1]
    return pl.pallas_call(
        body,
        out_shape=[
            jax.ShapeDtypeStruct((N_NODES, d2), jnp.float32),
            jax.ShapeDtypeStruct((N_NODES, d2), jnp.float32),
        ],
    )(accp, selfp, dis, b, w)


def _tc_final(accp, selfp, dis, b):
    def body(accp_ref, self_ref, dis_ref, b_ref, o_ref):
        z = (dis_ref[...] * (accp_ref[0] + accp_ref[1])
             + self_ref[...] + b_ref[...])
        m = jnp.max(z, axis=1, keepdims=True)
        e = jnp.exp(z - m)
        o_ref[...] = (z - m) - jnp.log(jnp.sum(e, axis=1, keepdims=True))

    d = selfp.shape[1]
    return pl.pallas_call(
        body,
        out_shape=jax.ShapeDtypeStruct((N_NODES, d), jnp.float32),
    )(accp, selfp, dis, b)


def kernel(x, edge_index, W1, b1, W2, b2, W3, b3):
    e3 = edge_index.reshape(2, NUM_WORKERS * NCHUNK, CHUNK)
    ones = jnp.ones((CHUNK, DEG_W), jnp.float32)

    degp = _sc_degree(e3, ones, jnp.zeros((ROWS_PER_TILE, DEG_W), jnp.float32))
    hp1, self1, dis = _tc_stage1(x, W1, degp)

    acc1 = _sc_aggregate(hp1, e3, jnp.zeros((ROWS_PER_TILE, 32), jnp.float32), 32)
    hp2, self2 = _tc_mid(acc1, self1, dis, b1.reshape(1, -1), W2)

    acc2 = _sc_aggregate(hp2, e3, jnp.zeros((ROWS_PER_TILE, 16), jnp.float32), 16)
    hp3, self3 = _tc_mid(acc2, self2, dis, b2.reshape(1, -1), W3)

    acc3 = _sc_aggregate(hp3, e3, jnp.zeros((ROWS_PER_TILE, 40), jnp.float32), 40)
    return _tc_final(acc3, self3, dis, b3.reshape(1, -1))
